# Initial kernel scaffold; baseline (speedup 1.0000x reference)
import jax
import jax.numpy as jnp
from jax.experimental import pallas as pl


def kernel(q, k, v, k_cache, v_cache, slot_mapping, block_tables, context_lens):
    raise NotImplementedError("write your pallas kernel here")


# speedup vs baseline: 6.5444x; 6.5444x over previous
import jax
import jax.numpy as jnp
from jax.experimental import pallas as pl
from jax.experimental.pallas import tpu as pltpu

B, H, KV, HD = 64, 32, 8, 128
BS, MAXC = 16, 2048
MB = MAXC // BS
G = H // KV
SCALE = HD ** -0.5
BPI = 8
T = BPI * BS
NSTEPS = MB // BPI
LANES = BPI * BS * KV
NEG = -1e30


def _attn_kernel(phys_ref, ctx_ref, q_ref, kn_ref, vn_ref, *refs):
    kc = refs[:BPI]
    vc = refs[BPI:2 * BPI]
    out_ref = refs[2 * BPI]
    acc_ref, m_ref, l_ref = refs[2 * BPI + 1:]

    b = pl.program_id(0)
    j = pl.program_id(1)
    ctx = ctx_ref[b]

    @pl.when(j == 0)
    def _init():
        m_ref[...] = jnp.full_like(m_ref, NEG)
        l_ref[...] = jnp.zeros_like(l_ref)
        acc_ref[...] = jnp.zeros_like(acc_ref)

    @pl.when(j * T < ctx)
    def _body():
        q = q_ref[0]
        parts = []
        for i in range(BPI):
            k2 = kc[i][0].reshape(BS * KV, HD)
            parts.append(jax.lax.dot_general(
                q, k2, (((1,), (1,)), ((), ())),
                preferred_element_type=jnp.float32))
        s = jnp.concatenate(parts, axis=1) * SCALE

        col = jax.lax.broadcasted_iota(jnp.int32, (H, LANES), 1)
        row = jax.lax.broadcasted_iota(jnp.int32, (H, LANES), 0)
        kv_of_col = col & (KV - 1)
        pos = j * T + (col >> 3)
        head_match = kv_of_col == (row >> 2)
        valid = head_match & (pos < ctx)
        is_new = head_match & (pos == ctx - 1)

        s_new = jax.lax.dot_general(
            q, kn_ref[0], (((1,), (1,)), ((), ())),
            preferred_element_type=jnp.float32) * SCALE
        s = jnp.where(is_new, jnp.tile(s_new, (1, LANES // KV)), s)
        s = jnp.where(valid, s, NEG)

        m_prev = m_ref[...]
        l_prev = l_ref[...]
        m_new = jnp.maximum(m_prev, jnp.max(s, axis=1, keepdims=True))
        corr = jnp.exp(m_prev - m_new)
        p = jnp.exp(s - m_new[:, :1])
        l_ref[...] = l_prev * corr + jnp.sum(p, axis=1, keepdims=True)
        m_ref[...] = m_new

        p_pv = jnp.where(is_new, 0.0, p)
        pv = jnp.zeros((H, HD), jnp.float32)
        for i in range(BPI):
            v2 = vc[i][0].reshape(BS * KV, HD)
            pv = pv + jax.lax.dot_general(
                p_pv[:, i * BS * KV:(i + 1) * BS * KV], v2,
                (((1,), (0,)), ((), ())),
                preferred_element_type=jnp.float32)
        p_new = jnp.sum(jnp.where(is_new, p, 0.0), axis=1, keepdims=True)
        vn_rep = jnp.repeat(vn_ref[0], G, axis=0)
        acc_ref[...] = acc_ref[...] * corr + pv + p_new * vn_rep

    @pl.when(j == NSTEPS - 1)
    def _finalize():
        out_ref[0, 0] = acc_ref[...] / l_ref[...]


def _paged_attn(q, k, v, k_cache, v_cache, block_tables, context_lens,
                interpret=False):
    last_page = (context_lens - 1) // BS
    page_pos = jnp.minimum(jnp.arange(MB, dtype=jnp.int32)[None, :],
                           last_page[:, None])
    phys = jnp.take_along_axis(block_tables, page_pos, axis=1)

    cache_spec = [
        pl.BlockSpec((1, BS, KV, HD),
                     lambda b, j, phys, ctx, i=i: (phys[b, j * BPI + i], 0, 0, 0))
        for i in range(BPI)
    ]
    grid_spec = pltpu.PrefetchScalarGridSpec(
        num_scalar_prefetch=2,
        grid=(B, NSTEPS),
        in_specs=[
            pl.BlockSpec((1, H, HD), lambda b, j, phys, ctx: (b, 0, 0)),
            pl.BlockSpec((1, KV, HD), lambda b, j, phys, ctx: (b, 0, 0)),
            pl.BlockSpec((1, KV, HD), lambda b, j, phys, ctx: (b, 0, 0)),
            *cache_spec,
            *cache_spec,
        ],
        out_specs=pl.BlockSpec((1, 1, H, HD),
                               lambda b, j, phys, ctx: (b, 0, 0, 0)),
        scratch_shapes=[
            pltpu.VMEM((H, HD), jnp.float32),
            pltpu.VMEM((H, 128), jnp.float32),
            pltpu.VMEM((H, 128), jnp.float32),
        ],
    )
    out = pl.pallas_call(
        _attn_kernel,
        grid_spec=grid_spec,
        out_shape=jax.ShapeDtypeStruct((B, 1, H, HD), jnp.float32),
        compiler_params=pltpu.CompilerParams(
            dimension_semantics=("parallel", "arbitrary"),
        ),
        name="paged_decode_attn",
        interpret=interpret,
    )(phys, context_lens,
      q, k, v,
      *([k_cache] * BPI), *([v_cache] * BPI))
    return out


def kernel(q, k, v, k_cache, v_cache, slot_mapping, block_tables, context_lens):
    del slot_mapping
    return _paged_attn(q, k, v, k_cache, v_cache, block_tables, context_lens)


# speedup vs baseline: 10.6227x; 1.6232x over previous
import jax
import jax.numpy as jnp
from jax.experimental import pallas as pl
from jax.experimental.pallas import tpu as pltpu

B, H, KV, HD = 64, 32, 8, 128
BS, MAXC = 16, 2048
MB = MAXC // BS
G = H // KV
SCALE = HD ** -0.5
BPI = 8
T = BPI * BS
LANES = BPI * BS * KV
NEG = -1e30


def _attn_kernel(phys_ref, ctx_ref, q_ref, kn_ref, vn_ref, kc_hbm, vc_hbm,
                 out_ref, kbuf, vbuf, ksem, vsem):
    b = pl.program_id(0)
    ctx = ctx_ref[b]
    nc = (ctx + T - 1) // T

    def issue(c, slot):
        base = c * BPI
        for i in range(BPI):
            pid = phys_ref[b, base + i]
            pltpu.make_async_copy(kc_hbm.at[pid], kbuf.at[slot, i],
                                  ksem.at[slot]).start()
            pltpu.make_async_copy(vc_hbm.at[pid], vbuf.at[slot, i],
                                  vsem.at[slot]).start()

    issue(0, 0)

    q = q_ref[0]
    s_new = jax.lax.dot_general(
        q, kn_ref[0], (((1,), (1,)), ((), ())),
        preferred_element_type=jnp.float32) * SCALE
    s_new_t = jnp.tile(s_new, (1, LANES // KV))
    vn_rep = jnp.repeat(vn_ref[0], G, axis=0)

    col = jax.lax.broadcasted_iota(jnp.int32, (H, LANES), 1)
    row = jax.lax.broadcasted_iota(jnp.int32, (H, LANES), 0)
    head_match = (col & (KV - 1)) == (row >> 2)
    tok = col >> 3

    def body(c, carry):
        m_p, l_p, acc_p = carry
        slot = jax.lax.rem(c, 2)

        @pl.when(c + 1 < nc)
        def _prefetch():
            issue(c + 1, jax.lax.rem(c + 1, 2))

        pltpu.make_async_copy(kc_hbm.at[0], kbuf.at[slot], ksem.at[slot]).wait()
        pltpu.make_async_copy(vc_hbm.at[0], vbuf.at[slot], vsem.at[slot]).wait()

        k2 = kbuf[slot].reshape(LANES, HD)
        s = jax.lax.dot_general(
            q, k2, (((1,), (1,)), ((), ())),
            preferred_element_type=jnp.float32) * SCALE

        rem = ctx - c * T
        valid = head_match & (tok < rem)
        is_new = head_match & (tok == rem - 1)
        s = jnp.where(is_new, s_new_t, s)
        s = jnp.where(valid, s, NEG)

        m_n = jnp.maximum(m_p, jnp.max(s, axis=1, keepdims=True))
        corr = jnp.exp(m_p - m_n)
        p = jnp.exp(s - m_n[:, :1])
        l_n = l_p * corr + jnp.sum(p, axis=1, keepdims=True)

        p_pv = jnp.where(is_new, 0.0, p)
        v2 = vbuf[slot].reshape(LANES, HD)
        pv = jax.lax.dot_general(
            p_pv, v2, (((1,), (0,)), ((), ())),
            preferred_element_type=jnp.float32)
        p_new = jnp.sum(jnp.where(is_new, p, 0.0), axis=1, keepdims=True)
        acc_n = acc_p * corr + pv + p_new * vn_rep
        return (m_n, l_n, acc_n)

    m0 = jnp.full((H, 128), NEG, jnp.float32)
    l0 = jnp.zeros((H, 128), jnp.float32)
    a0 = jnp.zeros((H, HD), jnp.float32)
    m_f, l_f, acc_f = jax.lax.fori_loop(0, nc, body, (m0, l0, a0))
    out_ref[0, 0] = acc_f / l_f


def _paged_attn(q, k, v, k_cache, v_cache, block_tables, context_lens,
                interpret=False):
    last_page = (context_lens - 1) // BS
    page_pos = jnp.minimum(jnp.arange(MB, dtype=jnp.int32)[None, :],
                           last_page[:, None])
    phys = jnp.take_along_axis(block_tables, page_pos, axis=1)

    grid_spec = pltpu.PrefetchScalarGridSpec(
        num_scalar_prefetch=2,
        grid=(B,),
        in_specs=[
            pl.BlockSpec((1, H, HD), lambda b, phys, ctx: (b, 0, 0)),
            pl.BlockSpec((1, KV, HD), lambda b, phys, ctx: (b, 0, 0)),
            pl.BlockSpec((1, KV, HD), lambda b, phys, ctx: (b, 0, 0)),
            pl.BlockSpec(memory_space=pl.ANY),
            pl.BlockSpec(memory_space=pl.ANY),
        ],
        out_specs=pl.BlockSpec((1, 1, H, HD),
                               lambda b, phys, ctx: (b, 0, 0, 0)),
        scratch_shapes=[
            pltpu.VMEM((2, BPI, BS, KV, HD), jnp.float32),
            pltpu.VMEM((2, BPI, BS, KV, HD), jnp.float32),
            pltpu.SemaphoreType.DMA((2,)),
            pltpu.SemaphoreType.DMA((2,)),
        ],
    )
    out = pl.pallas_call(
        _attn_kernel,
        grid_spec=grid_spec,
        out_shape=jax.ShapeDtypeStruct((B, 1, H, HD), jnp.float32),
        compiler_params=pltpu.CompilerParams(
            dimension_semantics=("arbitrary",),
        ),
        name="paged_decode_attn",
        interpret=interpret,
    )(phys, context_lens, q, k, v, k_cache, v_cache)
    return out


def kernel(q, k, v, k_cache, v_cache, slot_mapping, block_tables, context_lens):
    del slot_mapping
    return _paged_attn(q, k, v, k_cache, v_cache, block_tables, context_lens)
